# baseline (speedup 1.0000x reference)
import functools

import jax
import jax.numpy as jnp
from jax import lax
from jax.experimental import pallas as pl
from jax.experimental.pallas import tpu as pltpu


def _round_up(x, m):
    return (x + m - 1) // m * m


_SQRT_2_OVER_PI = 0.7978845608028654


def _gelu_tanh(x):
    return 0.5 * x * (1.0 + jnp.tanh(_SQRT_2_OVER_PI * (x + 0.044715 * x * x * x)))


def _transform_kernel(eps, x_ref, wt_ref, bt_ref, gamma_ref, beta_ref,
                      pooled_ref, ws_ref, bs_ref, h_ref, nsp_ref):
    xb = x_ref[...].astype(jnp.bfloat16)
    h = jnp.dot(xb, wt_ref[...], preferred_element_type=jnp.float32)
    h = h + bt_ref[...]
    h = _gelu_tanh(h)
    mean = jnp.mean(h, axis=-1, keepdims=True)
    cent = h - mean
    var = jnp.mean(cent * cent, axis=-1, keepdims=True)
    h = cent * lax.rsqrt(var + eps)
    h = h * gamma_ref[...] + beta_ref[...]
    h_ref[...] = h.T.astype(jnp.bfloat16)

    nsp = jnp.dot(pooled_ref[...], ws_ref[...],
                  preferred_element_type=jnp.float32) + bs_ref[...]
    nsp_ref[...] = nsp[None]


def _decoder_kernel(b, s, wdt_ref, ht_ref, bd_ref, out_ref):
    wb = wdt_ref[...].astype(jnp.bfloat16)
    res = jnp.dot(wb, ht_ref[...], preferred_element_type=jnp.float32)
    res = res + bd_ref[...]
    out_ref[...] = res.reshape(out_ref.shape)


def kernel(sequence_output, pooled_output, transform_dense_w,
           transform_dense_b, ln_gamma, ln_beta, decoder_w, decoder_b,
           seq_rel_w, seq_rel_b, *, layer_norm_eps=1e-12):
    B, S, H = sequence_output.shape
    V = decoder_w.shape[1]
    L = seq_rel_w.shape[1]
    N = B * S

    x = sequence_output.reshape(N, H)

    tn = min(1024, _round_up(N, 8))
    n_tiles = pl.cdiv(N, tn)

    wt_b = transform_dense_w.astype(jnp.bfloat16)
    bt = transform_dense_b.reshape(1, H).astype(jnp.float32)
    gamma = ln_gamma.reshape(1, H).astype(jnp.float32)
    beta = ln_beta.reshape(1, H).astype(jnp.float32)

    b_pad = _round_up(B, 8)
    l_pad = _round_up(L, 128)
    pooled = pooled_output.astype(jnp.float32)
    if b_pad != B:
        pooled = jnp.pad(pooled, ((0, b_pad - B), (0, 0)))
    ws = seq_rel_w.astype(jnp.float32)
    bs = seq_rel_b.astype(jnp.float32)
    if l_pad != L:
        ws = jnp.pad(ws, ((0, 0), (0, l_pad - L)))
        bs = jnp.pad(bs, ((0, l_pad - L),))
    bs = bs.reshape(1, l_pad)

    h_t, nsp_all = pl.pallas_call(
        lambda *refs: _transform_kernel(float(layer_norm_eps), *refs),
        out_shape=(
            jax.ShapeDtypeStruct((H, N), jnp.bfloat16),
            jax.ShapeDtypeStruct((n_tiles, b_pad, l_pad), jnp.float32),
        ),
        grid=(n_tiles,),
        in_specs=[
            pl.BlockSpec((tn, H), lambda i: (i, 0)),
            pl.BlockSpec((H, H), lambda i: (0, 0)),
            pl.BlockSpec((1, H), lambda i: (0, 0)),
            pl.BlockSpec((1, H), lambda i: (0, 0)),
            pl.BlockSpec((1, H), lambda i: (0, 0)),
            pl.BlockSpec((b_pad, H), lambda i: (0, 0)),
            pl.BlockSpec((H, l_pad), lambda i: (0, 0)),
            pl.BlockSpec((1, l_pad), lambda i: (0, 0)),
        ],
        out_specs=(
            pl.BlockSpec((H, tn), lambda i: (0, i)),
            pl.BlockSpec((1, b_pad, l_pad), lambda i: (i, 0, 0)),
        ),
        compiler_params=pltpu.CompilerParams(
            dimension_semantics=("parallel",),
            vmem_limit_bytes=32 * 1024 * 1024),
    )(x, wt_b, bt, gamma, beta, pooled, ws, bs)

    seq_relationship_score = nsp_all[0, :B, :L]

    wd_t = decoder_w.T
    bd_col = decoder_b.reshape(V, 1)

    tv = min(1024, _round_up(V, 8))
    v_tiles = pl.cdiv(V, tv)

    dec = functools.partial(_decoder_kernel, B, S)

    out_t = pl.pallas_call(
        dec,
        out_shape=jax.ShapeDtypeStruct((V, B, S), jnp.float32),
        grid=(v_tiles,),
        in_specs=[
            pl.BlockSpec((tv, H), lambda j: (j, 0)),
            pl.BlockSpec((H, N), lambda j: (0, 0)),
            pl.BlockSpec((tv, 1), lambda j: (j, 0)),
        ],
        out_specs=pl.BlockSpec((tv, B, S), lambda j: (j, 0, 0)),
        compiler_params=pltpu.CompilerParams(
            dimension_semantics=("parallel",),
            vmem_limit_bytes=60 * 1024 * 1024),
    )(wd_t, h_t, bd_col)

    prediction_scores = jnp.transpose(out_t, (1, 2, 0))
    return prediction_scores, seq_relationship_score


# speedup vs baseline: 1.0228x; 1.0228x over previous
import jax
import jax.numpy as jnp
from jax import lax
from jax.experimental import pallas as pl
from jax.experimental.pallas import tpu as pltpu


def _round_up(x, m):
    return (x + m - 1) // m * m


_SQRT_2_OVER_PI = 0.7978845608028654


def _gelu_tanh(x):
    return 0.5 * x * (1.0 + jnp.tanh(_SQRT_2_OVER_PI * (x + 0.044715 * x * x * x)))


def _transform_kernel(eps, x_ref, wt_ref, bt_ref, gamma_ref, beta_ref,
                      pooled_ref, ws_ref, bs_ref, h_ref, nsp_ref):
    xb = x_ref[...].astype(jnp.bfloat16)
    h = jnp.dot(xb, wt_ref[...], preferred_element_type=jnp.float32)
    h = h + bt_ref[...]
    h = _gelu_tanh(h)
    mean = jnp.mean(h, axis=-1, keepdims=True)
    cent = h - mean
    var = jnp.mean(cent * cent, axis=-1, keepdims=True)
    h = cent * lax.rsqrt(var + eps)
    h = h * gamma_ref[...] + beta_ref[...]
    h_ref[...] = h.T.astype(jnp.bfloat16)

    nsp = jnp.dot(pooled_ref[...], ws_ref[...],
                  preferred_element_type=jnp.float32) + bs_ref[...]
    nsp_ref[...] = nsp[None]


def _decoder_kernel(wdt_ref, ht_ref, bd_ref, out_ref):
    wb = wdt_ref[...].astype(jnp.bfloat16)
    res = jnp.dot(wb, ht_ref[...], preferred_element_type=jnp.float32)
    res = res + bd_ref[...]
    out_ref[...] = res.reshape(out_ref.shape)


def kernel(sequence_output, pooled_output, transform_dense_w,
           transform_dense_b, ln_gamma, ln_beta, decoder_w, decoder_b,
           seq_rel_w, seq_rel_b, *, layer_norm_eps=1e-12):
    B, S, H = sequence_output.shape
    V = decoder_w.shape[1]
    L = seq_rel_w.shape[1]
    N = B * S

    x = sequence_output.reshape(N, H)

    tn = min(1024, _round_up(N, 8))
    n_tiles = pl.cdiv(N, tn)

    wt_b = transform_dense_w.astype(jnp.bfloat16)
    bt = transform_dense_b.reshape(1, H).astype(jnp.float32)
    gamma = ln_gamma.reshape(1, H).astype(jnp.float32)
    beta = ln_beta.reshape(1, H).astype(jnp.float32)

    b_pad = _round_up(B, 8)
    l_pad = _round_up(L, 128)
    pooled = pooled_output.astype(jnp.float32)
    if b_pad != B:
        pooled = jnp.pad(pooled, ((0, b_pad - B), (0, 0)))
    ws = seq_rel_w.astype(jnp.float32)
    bs = seq_rel_b.astype(jnp.float32)
    if l_pad != L:
        ws = jnp.pad(ws, ((0, 0), (0, l_pad - L)))
        bs = jnp.pad(bs, ((0, l_pad - L),))
    bs = bs.reshape(1, l_pad)

    h_t, nsp_all = pl.pallas_call(
        lambda *refs: _transform_kernel(float(layer_norm_eps), *refs),
        out_shape=(
            jax.ShapeDtypeStruct((H, N), jnp.bfloat16),
            jax.ShapeDtypeStruct((n_tiles, b_pad, l_pad), jnp.float32),
        ),
        grid=(n_tiles,),
        in_specs=[
            pl.BlockSpec((tn, H), lambda i: (i, 0)),
            pl.BlockSpec((H, H), lambda i: (0, 0)),
            pl.BlockSpec((1, H), lambda i: (0, 0)),
            pl.BlockSpec((1, H), lambda i: (0, 0)),
            pl.BlockSpec((1, H), lambda i: (0, 0)),
            pl.BlockSpec((b_pad, H), lambda i: (0, 0)),
            pl.BlockSpec((H, l_pad), lambda i: (0, 0)),
            pl.BlockSpec((1, l_pad), lambda i: (0, 0)),
        ],
        out_specs=(
            pl.BlockSpec((H, tn), lambda i: (0, i)),
            pl.BlockSpec((1, b_pad, l_pad), lambda i: (i, 0, 0)),
        ),
        compiler_params=pltpu.CompilerParams(
            dimension_semantics=("parallel",),
            vmem_limit_bytes=32 * 1024 * 1024),
    )(x, wt_b, bt, gamma, beta, pooled, ws, bs)

    seq_relationship_score = nsp_all[0, :B, :L]

    wd_t = decoder_w.T
    bd_col = decoder_b.reshape(V, 1)

    tv = min(1024, _round_up(V, 8))
    v_tiles = pl.cdiv(V, tv)

    out_t = pl.pallas_call(
        _decoder_kernel,
        out_shape=jax.ShapeDtypeStruct((V, B, S), jnp.float32),
        grid=(v_tiles,),
        in_specs=[
            pl.BlockSpec((tv, H), lambda j: (j, 0)),
            pl.BlockSpec((H, N), lambda j: (0, 0)),
            pl.BlockSpec((tv, 1), lambda j: (j, 0)),
        ],
        out_specs=pl.BlockSpec((tv, B, S), lambda j: (j, 0, 0)),
        compiler_params=pltpu.CompilerParams(
            dimension_semantics=("parallel",),
            vmem_limit_bytes=60 * 1024 * 1024),
    )(wd_t, h_t, bd_col)

    prediction_scores = jnp.transpose(out_t, (1, 2, 0))
    return prediction_scores, seq_relationship_score
